# baseline (speedup 1.0000x reference)
import functools

import jax
import jax.numpy as jnp
from jax import lax
from jax.experimental import pallas as pl
from jax.experimental.pallas import tpu as pltpu
from jax.experimental.pallas import tpu_sc as plsc

N = 10000
NPAD = 10240
NROW, NCOL = 80, 128
Q = 2500
QPAD = 2560
QBLK = 256
NQB = QPAD // QBLK
K = 64
D = 256
DOUT = 512
E = K * QPAD
R2 = 0.2 * 0.2
FAR = 3.0e18
NCHUNK = 1024


def _fps_kernel(pos3_ref, poss_ref, sel_ref):
    px = pos3_ref[0]
    py = pos3_ref[1]
    pz = pos3_ref[2]
    fi = (lax.broadcasted_iota(jnp.int32, (NROW, NCOL), 0) * NCOL
          + lax.broadcasted_iota(jnp.int32, (NROW, NCOL), 1))
    fif = fi.astype(jnp.float32)
    inf = jnp.float32(jnp.inf)
    mind0 = jnp.where(fi < N, inf, -inf)
    sel_ref[0] = 0

    def body(i, carry):
        cur, mind = carry
        cx = poss_ref[0, cur]
        cy = poss_ref[1, cur]
        cz = poss_ref[2, cur]
        dx = px - cx
        dy = py - cy
        dz = pz - cz
        d = (dx * dx + dz * dz) + dy * dy
        mind = jnp.minimum(mind, d)
        m = jnp.max(mind)
        nxt = jnp.min(jnp.where(mind == m, fif, inf)).astype(jnp.int32)
        sel_ref[i] = nxt
        return (nxt, mind)

    lax.fori_loop(1, Q, body, (jnp.int32(0), mind0))


def _run_fps(pos3):
    return pl.pallas_call(
        _fps_kernel,
        out_shape=jax.ShapeDtypeStruct((Q,), jnp.int32),
        in_specs=[
            pl.BlockSpec(memory_space=pltpu.VMEM),
            pl.BlockSpec(memory_space=pltpu.SMEM),
        ],
        out_specs=pl.BlockSpec(memory_space=pltpu.SMEM),
    )(pos3, pos3.reshape(3, NPAD))


def _radius_kernel(posq_ref, pos3_ref, w1b_ref, nbrT_ref, vmT_ref, v_ref,
                   kiter_ref, keys_ref):
    qi = pl.program_id(0)
    qx = posq_ref[:, 0:1]
    qy = posq_ref[:, 1:2]
    qz = posq_ref[:, 2:3]

    v_ref[...] = (qx * w1b_ref[0:1, :] + qy * w1b_ref[1:2, :]
                  + qz * w1b_ref[2:3, :])

    cnt = jnp.zeros((QBLK,), jnp.int32)
    for c in range(0, NPAD, NCHUNK):
        px = pos3_ref[0:1, c:c + NCHUNK]
        py = pos3_ref[1:2, c:c + NCHUNK]
        pz = pos3_ref[2:3, c:c + NCHUNK]
        dx = qx - px
        dy = qy - py
        dz = qz - pz
        d2 = dx * dx + dy * dy + dz * dz
        mask = d2 <= R2
        ji = c + lax.broadcasted_iota(jnp.int32, (QBLK, NCHUNK), 1)
        keys_ref[:, c:c + NCHUNK] = jnp.where(mask, ji, jnp.int32(N))
        cnt = cnt + jnp.sum(mask.astype(jnp.int32), axis=1)

    kiter = jnp.max(jnp.minimum(cnt, K))
    kiter_ref[qi] = kiter
    nbrT_ref[...] = jnp.zeros((K, QBLK), jnp.int32)
    vmT_ref[...] = jnp.zeros((K, QBLK), jnp.float32)

    def body(k, _):
        @pl.when(k < kiter)
        def _():
            m = jnp.full((QBLK,), jnp.int32(2**30))
            for c in range(0, NPAD, NCHUNK):
                m = jnp.minimum(m, jnp.min(keys_ref[:, c:c + NCHUNK], axis=1))
            valid = m < N
            nbrT_ref[pl.ds(k, 1), :] = jnp.where(valid, m, 0).reshape(1, QBLK)
            vmT_ref[pl.ds(k, 1), :] = valid.astype(jnp.float32).reshape(1, QBLK)
            mcol = m.reshape(QBLK, 1)
            for c in range(0, NPAD, NCHUNK):
                kk = keys_ref[:, c:c + NCHUNK]
                keys_ref[:, c:c + NCHUNK] = jnp.where(kk == mcol,
                                                      jnp.int32(20000), kk)
        return 0

    lax.fori_loop(0, K, body, 0)


def _run_radius(posq_pad, pos3, w1b):
    return pl.pallas_call(
        _radius_kernel,
        grid=(NQB,),
        out_shape=(
            jax.ShapeDtypeStruct((K, QPAD), jnp.int32),
            jax.ShapeDtypeStruct((K, QPAD), jnp.float32),
            jax.ShapeDtypeStruct((QPAD, D), jnp.float32),
            jax.ShapeDtypeStruct((NQB,), jnp.int32),
        ),
        in_specs=[
            pl.BlockSpec((QBLK, 3), lambda qi: (qi, 0)),
            pl.BlockSpec(memory_space=pltpu.VMEM),
            pl.BlockSpec(memory_space=pltpu.VMEM),
        ],
        out_specs=(
            pl.BlockSpec((K, QBLK), lambda qi: (0, qi)),
            pl.BlockSpec((K, QBLK), lambda qi: (0, qi)),
            pl.BlockSpec((QBLK, D), lambda qi: (qi, 0)),
            pl.BlockSpec(memory_space=pltpu.SMEM),
        ),
        scratch_shapes=[pltpu.VMEM((QBLK, NPAD), jnp.int32)],
    )(posq_pad, pos3, w1b)


def _prep_kernel(x_ref, posp_ref, w1a_ref, w1b_ref, b1_ref, u_ref):
    u = jnp.dot(x_ref[...], w1a_ref[...], preferred_element_type=jnp.float32)
    u = (u + posp_ref[:, 0:1] * w1b_ref[0:1, :]
         + posp_ref[:, 1:2] * w1b_ref[1:2, :]
         + posp_ref[:, 2:3] * w1b_ref[2:3, :] + b1_ref[...])
    u_ref[...] = u


def _run_prep(x_pad, pos_pad, w1a, w1b, b1r):
    blk = 1024
    return pl.pallas_call(
        _prep_kernel,
        grid=(NPAD // blk,),
        out_shape=jax.ShapeDtypeStruct((NPAD, D), jnp.float32),
        in_specs=[
            pl.BlockSpec((blk, D), lambda i: (i, 0)),
            pl.BlockSpec((blk, 3), lambda i: (i, 0)),
            pl.BlockSpec((D, D), lambda i: (0, 0)),
            pl.BlockSpec((3, D), lambda i: (0, 0)),
            pl.BlockSpec((1, D), lambda i: (0, 0)),
        ],
        out_specs=pl.BlockSpec((blk, D), lambda i: (i, 0)),
    )(x_pad, pos_pad, w1a, w1b, b1r)


NC = 2
NS = 16
DH = D // NC
BPW = E // NS
WIN = 128
NWIN = BPW // WIN


def _run_sc_gather(u, idx):
    mesh = plsc.VectorSubcoreMesh(core_axis_name="c", subcore_axis_name="s")

    @functools.partial(
        pl.kernel,
        out_type=jax.ShapeDtypeStruct((E, D), jnp.float32),
        mesh=mesh,
        scratch_types=[
            pltpu.VMEM((BPW,), jnp.int32),
            pltpu.VMEM((2, WIN, DH), jnp.float32),
            pltpu.VMEM_SHARED((NPAD, DH), jnp.float32),
            pltpu.SemaphoreType.DMA,
            pltpu.SemaphoreType.DMA,
            pltpu.SemaphoreType.DMA,
            pltpu.SemaphoreType.DMA,
            pltpu.SemaphoreType.DMA,
        ],
    )
    def kgather(u_hbm, i_hbm, g_hbm, idx_v, bufs, ushr, sem0, ga, gb, sa, sb):
        gsem = (ga, gb)
        ssem = (sa, sb)
        sid = lax.axis_index("s")
        cid = lax.axis_index("c")
        base = sid * BPW

        @pl.when(sid == 0)
        def _():
            pltpu.async_copy(
                u_hbm.at[pl.ds(0, NPAD), pl.ds(cid * DH, DH)], ushr,
                sem0).wait()

        plsc.subcore_barrier()
        pltpu.async_copy(i_hbm.at[pl.ds(base, BPW)], idx_v, sem0).wait()

        def gather_copy(w, b):
            return pltpu.make_async_copy(
                ushr.at[idx_v.at[pl.ds(w * WIN, WIN)]], bufs.at[b], gsem[b])

        def store_copy(w, b):
            return pltpu.make_async_copy(
                bufs.at[b],
                g_hbm.at[pl.ds(base + w * WIN, WIN), pl.ds(cid * DH, DH)],
                ssem[b])

        gather_copy(0, 0).start()

        @pl.loop(0, NWIN, step=2)
        def _(w0):
            for b in range(2):
                w = w0 + b
                nb = 1 - b

                @pl.when(w + 1 < NWIN)
                def _():
                    @pl.when(w >= 1)
                    def _():
                        store_copy(w - 1, nb).wait()

                    gather_copy(w + 1, nb).start()

                gather_copy(w, b).wait()
                store_copy(w, b).start()

        store_copy(NWIN - 2, 0).wait()
        store_copy(NWIN - 1, 1).wait()

    return kgather(u, idx)


def _mlp_kernel(kiter_ref, g_ref, v_ref, vm_ref, w2_ref, b2_ref, w3_ref,
                b3_ref, out_ref, acc_ref):
    qi = pl.program_id(0)
    k = pl.program_id(1)
    kiter = kiter_ref[qi]

    @pl.when(k == 0)
    def _():
        acc_ref[...] = jnp.full((QBLK, DOUT), -jnp.inf, jnp.float32)

    @pl.when(k < kiter)
    def _():
        h1 = jnp.maximum(g_ref[0] - v_ref[...], 0.0)
        h2 = jnp.dot(h1.astype(jnp.bfloat16), w2_ref[...],
                     preferred_element_type=jnp.float32)
        h2 = jnp.maximum(h2 + b2_ref[...], 0.0)
        h3 = jnp.dot(h2.astype(jnp.bfloat16), w3_ref[...],
                     preferred_element_type=jnp.float32)
        h3 = jnp.maximum(h3 + b3_ref[...], 0.0)
        vm = vm_ref[...].reshape(QBLK, 1)
        acc_ref[...] = jnp.maximum(acc_ref[...],
                                   jnp.where(vm > 0.5, h3, -jnp.inf))

    @pl.when(k == K - 1)
    def _():
        a = acc_ref[...]
        out_ref[...] = jnp.where(jnp.isfinite(a), a, 0.0)


def _run_mlp(kiter, g3, v, vmf, w2, b2r, w3, b3r):
    return pl.pallas_call(
        _mlp_kernel,
        grid=(NQB, K),
        out_shape=jax.ShapeDtypeStruct((QPAD, DOUT), jnp.float32),
        in_specs=[
            pl.BlockSpec(memory_space=pltpu.SMEM),
            pl.BlockSpec((1, QBLK, D), lambda qi, k: (k, qi, 0)),
            pl.BlockSpec((QBLK, D), lambda qi, k: (qi, 0)),
            pl.BlockSpec((QBLK,), lambda qi, k: (k * NQB + qi,)),
            pl.BlockSpec((D, D), lambda qi, k: (0, 0)),
            pl.BlockSpec((1, D), lambda qi, k: (0, 0)),
            pl.BlockSpec((D, DOUT), lambda qi, k: (0, 0)),
            pl.BlockSpec((1, DOUT), lambda qi, k: (0, 0)),
        ],
        out_specs=pl.BlockSpec((QBLK, DOUT), lambda qi, k: (qi, 0)),
        scratch_shapes=[pltpu.VMEM((QBLK, DOUT), jnp.float32)],
    )(kiter, g3, v, vmf, w2, b2r, w3, b3r)


def kernel(x, pos, batch, W1, b1, W2, b2, W3, b3):
    pos = pos.astype(x.dtype)

    pos_pad = jnp.pad(pos, ((0, NPAD - N), (0, 0)), constant_values=FAR)
    pos3 = pos_pad.T.reshape(3, NROW, NCOL)
    pos3r = pos_pad.T

    sel = _run_fps(pos3)
    pos_q = jnp.take(pos, sel, axis=0)

    posq_pad = jnp.pad(pos_q, ((0, QPAD - Q), (0, 0)), constant_values=FAR)
    w1a = W1[:D, :]
    w1b = W1[D:, :]

    nbrT, vmT, v, kiter = _run_radius(posq_pad, pos3r, w1b)

    x_pad = jnp.pad(x, ((0, NPAD - N), (0, 0)))
    u = _run_prep(x_pad, pos_pad, w1a, w1b, b1.reshape(1, D))

    g = _run_sc_gather(u, nbrT.reshape(E))

    out_pad = _run_mlp(kiter, g.reshape(K, QPAD, D), v, vmT.reshape(E),
                       W2.astype(jnp.bfloat16), b2.reshape(1, D),
                       W3.astype(jnp.bfloat16), b3.reshape(1, DOUT))

    out = out_pad[:Q]
    batch_q = jnp.take(batch, sel, axis=0)
    return out, pos_q, batch_q


# speedup vs baseline: 1.1814x; 1.1814x over previous
import functools

import jax
import jax.numpy as jnp
from jax import lax
from jax.experimental import pallas as pl
from jax.experimental.pallas import tpu as pltpu
from jax.experimental.pallas import tpu_sc as plsc

N = 10000
NPAD = 10240
NROW, NCOL = 80, 128
Q = 2500
QPAD = 2560
QBLK = 256
NQB = QPAD // QBLK
K = 64
D = 256
DOUT = 512
E = K * QPAD
R2 = 0.2 * 0.2
FAR = 3.0e18
NCHUNK = 1024


def _fps_kernel(pos3_ref, poss_ref, sel_ref):
    px = pos3_ref[0]
    py = pos3_ref[1]
    pz = pos3_ref[2]
    fi = (lax.broadcasted_iota(jnp.int32, (NROW, NCOL), 0) * NCOL
          + lax.broadcasted_iota(jnp.int32, (NROW, NCOL), 1))
    fif = fi.astype(jnp.float32)
    inf = jnp.float32(jnp.inf)
    mind0 = jnp.where(fi < N, inf, -inf)
    sel_ref[0] = 0

    def body(i, carry):
        cur, mind = carry
        cx = poss_ref[0, cur]
        cy = poss_ref[1, cur]
        cz = poss_ref[2, cur]
        dx = px - cx
        dy = py - cy
        dz = pz - cz
        d = (dx * dx + dz * dz) + dy * dy
        mind = jnp.minimum(mind, d)
        m = jnp.max(mind)
        nxt = jnp.min(jnp.where(mind == m, fif, inf)).astype(jnp.int32)
        sel_ref[i] = nxt
        return (nxt, mind)

    lax.fori_loop(1, Q, body, (jnp.int32(0), mind0))


def _run_fps(pos3):
    return pl.pallas_call(
        _fps_kernel,
        out_shape=jax.ShapeDtypeStruct((Q,), jnp.int32),
        in_specs=[
            pl.BlockSpec(memory_space=pltpu.VMEM),
            pl.BlockSpec(memory_space=pltpu.SMEM),
        ],
        out_specs=pl.BlockSpec(memory_space=pltpu.SMEM),
    )(pos3, pos3.reshape(3, NPAD))


def _radius_kernel(posq_ref, pos3_ref, w1b_ref, nbrT_ref, cntf_ref, v_ref,
                   kiter_ref, keys_ref):
    qi = pl.program_id(0)
    qx = posq_ref[:, 0:1]
    qy = posq_ref[:, 1:2]
    qz = posq_ref[:, 2:3]

    v_ref[...] = (qx * w1b_ref[0:1, :] + qy * w1b_ref[1:2, :]
                  + qz * w1b_ref[2:3, :])

    cnt = jnp.zeros((QBLK,), jnp.int32)
    m0 = jnp.full((QBLK,), jnp.int32(2**30))
    for c in range(0, NPAD, NCHUNK):
        px = pos3_ref[0:1, c:c + NCHUNK]
        py = pos3_ref[1:2, c:c + NCHUNK]
        pz = pos3_ref[2:3, c:c + NCHUNK]
        dx = qx - px
        dy = qy - py
        dz = qz - pz
        d2 = dx * dx + dy * dy + dz * dz
        mask = d2 <= R2
        ji = c + lax.broadcasted_iota(jnp.int32, (QBLK, NCHUNK), 1)
        keys = jnp.where(mask, ji, jnp.int32(N))
        keys_ref[:, c:c + NCHUNK] = keys
        cnt = cnt + jnp.sum(mask.astype(jnp.int32), axis=1)
        m0 = jnp.minimum(m0, jnp.min(keys, axis=1))

    cntf_ref[...] = cnt.astype(jnp.float32)
    kiter = jnp.max(jnp.minimum(cnt, K))
    kiter_ref[qi] = kiter
    nbrT_ref[...] = jnp.zeros((K, QBLK), jnp.int32)

    def body(k, m):
        valid = m < N
        nbrT_ref[pl.ds(k, 1), :] = jnp.where(valid, m, 0).reshape(1, QBLK)
        mcol = m.reshape(QBLK, 1)
        nm = jnp.full((QBLK,), jnp.int32(2**30))
        for c in range(0, NPAD, NCHUNK):
            kk = keys_ref[:, c:c + NCHUNK]
            kk = jnp.where(kk == mcol, jnp.int32(20000), kk)
            keys_ref[:, c:c + NCHUNK] = kk
            nm = jnp.minimum(nm, jnp.min(kk, axis=1))
        return nm

    lax.fori_loop(0, kiter, body, m0)


def _run_radius(posq_pad, pos3, w1b):
    return pl.pallas_call(
        _radius_kernel,
        grid=(NQB,),
        out_shape=(
            jax.ShapeDtypeStruct((K, QPAD), jnp.int32),
            jax.ShapeDtypeStruct((QPAD,), jnp.float32),
            jax.ShapeDtypeStruct((QPAD, D), jnp.float32),
            jax.ShapeDtypeStruct((NQB,), jnp.int32),
        ),
        in_specs=[
            pl.BlockSpec((QBLK, 3), lambda qi: (qi, 0)),
            pl.BlockSpec(memory_space=pltpu.VMEM),
            pl.BlockSpec(memory_space=pltpu.VMEM),
        ],
        out_specs=(
            pl.BlockSpec((K, QBLK), lambda qi: (0, qi)),
            pl.BlockSpec((QBLK,), lambda qi: (qi,)),
            pl.BlockSpec((QBLK, D), lambda qi: (qi, 0)),
            pl.BlockSpec(memory_space=pltpu.SMEM),
        ),
        scratch_shapes=[pltpu.VMEM((QBLK, NPAD), jnp.int32)],
    )(posq_pad, pos3, w1b)


def _prep_kernel(x_ref, posp_ref, w1a_ref, w1b_ref, b1_ref, u_ref):
    u = jnp.dot(x_ref[...], w1a_ref[...], preferred_element_type=jnp.float32)
    u = (u + posp_ref[:, 0:1] * w1b_ref[0:1, :]
         + posp_ref[:, 1:2] * w1b_ref[1:2, :]
         + posp_ref[:, 2:3] * w1b_ref[2:3, :] + b1_ref[...])
    u_ref[...] = u


def _run_prep(x_pad, pos_pad, w1a, w1b, b1r):
    blk = 1024
    return pl.pallas_call(
        _prep_kernel,
        grid=(NPAD // blk,),
        out_shape=jax.ShapeDtypeStruct((NPAD, D), jnp.float32),
        in_specs=[
            pl.BlockSpec((blk, D), lambda i: (i, 0)),
            pl.BlockSpec((blk, 3), lambda i: (i, 0)),
            pl.BlockSpec((D, D), lambda i: (0, 0)),
            pl.BlockSpec((3, D), lambda i: (0, 0)),
            pl.BlockSpec((1, D), lambda i: (0, 0)),
        ],
        out_specs=pl.BlockSpec((blk, D), lambda i: (i, 0)),
    )(x_pad, pos_pad, w1a, w1b, b1r)


NC = 2
NS = 16
DH = D // NC
BPW = E // NS
WIN = 128
NWIN = BPW // WIN


def _run_sc_gather(u, idx):
    mesh = plsc.VectorSubcoreMesh(core_axis_name="c", subcore_axis_name="s")

    @functools.partial(
        pl.kernel,
        out_type=jax.ShapeDtypeStruct((E, D), jnp.float32),
        mesh=mesh,
        scratch_types=[
            pltpu.VMEM((BPW,), jnp.int32),
            pltpu.VMEM((2, WIN, DH), jnp.float32),
            pltpu.VMEM_SHARED((NPAD, DH), jnp.float32),
            pltpu.SemaphoreType.DMA,
            pltpu.SemaphoreType.DMA,
            pltpu.SemaphoreType.DMA,
            pltpu.SemaphoreType.DMA,
            pltpu.SemaphoreType.DMA,
        ],
    )
    def kgather(u_hbm, i_hbm, g_hbm, idx_v, bufs, ushr, sem0, ga, gb, sa, sb):
        gsem = (ga, gb)
        ssem = (sa, sb)
        sid = lax.axis_index("s")
        cid = lax.axis_index("c")
        base = sid * BPW

        @pl.when(sid == 0)
        def _():
            pltpu.async_copy(
                u_hbm.at[pl.ds(0, NPAD), pl.ds(cid * DH, DH)], ushr,
                sem0).wait()

        plsc.subcore_barrier()
        pltpu.async_copy(i_hbm.at[pl.ds(base, BPW)], idx_v, sem0).wait()

        def gather_copy(w, b):
            return pltpu.make_async_copy(
                ushr.at[idx_v.at[pl.ds(w * WIN, WIN)]], bufs.at[b], gsem[b])

        def store_copy(w, b):
            return pltpu.make_async_copy(
                bufs.at[b],
                g_hbm.at[pl.ds(base + w * WIN, WIN), pl.ds(cid * DH, DH)],
                ssem[b])

        gather_copy(0, 0).start()

        @pl.loop(0, NWIN, step=2)
        def _(w0):
            for b in range(2):
                w = w0 + b
                nb = 1 - b

                @pl.when(w + 1 < NWIN)
                def _():
                    @pl.when(w >= 1)
                    def _():
                        store_copy(w - 1, nb).wait()

                    gather_copy(w + 1, nb).start()

                gather_copy(w, b).wait()
                store_copy(w, b).start()

        store_copy(NWIN - 2, 0).wait()
        store_copy(NWIN - 1, 1).wait()

    return kgather(u, idx)


KB = 4
NKB = K // KB


def _mlp_kernel(kiter_ref, g_ref, v_ref, cnt_ref, w2_ref, b2_ref, w3_ref,
                b3_ref, out_ref, acc_ref):
    qi = pl.program_id(0)
    kb = pl.program_id(1)
    kiter = kiter_ref[qi]

    @pl.when(kb == 0)
    def _():
        acc_ref[...] = jnp.full((QBLK, DOUT), -jnp.inf, jnp.float32)

    @pl.when(kb * KB < kiter)
    def _():
        vb = v_ref[...]
        g2 = g_ref[0].reshape(KB * QBLK, D)
        vt = jnp.concatenate([vb, vb, vb, vb], axis=0)
        h1 = jnp.maximum(g2 - vt, 0.0)
        h2 = jnp.dot(h1.astype(jnp.bfloat16), w2_ref[...],
                     preferred_element_type=jnp.float32)
        h2 = jnp.maximum(h2 + b2_ref[...], 0.0)
        h3 = jnp.dot(h2.astype(jnp.bfloat16), w3_ref[...],
                     preferred_element_type=jnp.float32)
        h3 = jnp.maximum(h3 + b3_ref[...], 0.0)
        cnt = cnt_ref[...].reshape(QBLK, 1)
        a = acc_ref[...]
        for j in range(KB):
            kf = (kb * KB + j).astype(jnp.float32)
            h3j = h3[j * QBLK:(j + 1) * QBLK, :]
            a = jnp.maximum(a, jnp.where(kf < cnt, h3j, -jnp.inf))
        acc_ref[...] = a

    @pl.when(kb == NKB - 1)
    def _():
        a = acc_ref[...]
        out_ref[...] = jnp.where(jnp.isfinite(a), a, 0.0)


def _run_mlp(kiter, g4, v, cntf, w2, b2r, w3, b3r):
    return pl.pallas_call(
        _mlp_kernel,
        grid=(NQB, NKB),
        out_shape=jax.ShapeDtypeStruct((QPAD, DOUT), jnp.float32),
        in_specs=[
            pl.BlockSpec(memory_space=pltpu.SMEM),
            pl.BlockSpec((1, KB, QBLK, D), lambda qi, kb: (kb, 0, qi, 0)),
            pl.BlockSpec((QBLK, D), lambda qi, kb: (qi, 0)),
            pl.BlockSpec((QBLK,), lambda qi, kb: (qi,)),
            pl.BlockSpec((D, D), lambda qi, kb: (0, 0)),
            pl.BlockSpec((1, D), lambda qi, kb: (0, 0)),
            pl.BlockSpec((D, DOUT), lambda qi, kb: (0, 0)),
            pl.BlockSpec((1, DOUT), lambda qi, kb: (0, 0)),
        ],
        out_specs=pl.BlockSpec((QBLK, DOUT), lambda qi, kb: (qi, 0)),
        scratch_shapes=[pltpu.VMEM((QBLK, DOUT), jnp.float32)],
    )(kiter, g4, v, cntf, w2, b2r, w3, b3r)


def kernel(x, pos, batch, W1, b1, W2, b2, W3, b3):
    pos = pos.astype(x.dtype)

    pos_pad = jnp.pad(pos, ((0, NPAD - N), (0, 0)), constant_values=FAR)
    pos3 = pos_pad.T.reshape(3, NROW, NCOL)
    pos3r = pos_pad.T

    sel = _run_fps(pos3)
    pos_q = jnp.take(pos, sel, axis=0)

    posq_pad = jnp.pad(pos_q, ((0, QPAD - Q), (0, 0)), constant_values=FAR)
    w1a = W1[:D, :]
    w1b = W1[D:, :]

    nbrT, cntf, v, kiter = _run_radius(posq_pad, pos3r, w1b)

    x_pad = jnp.pad(x, ((0, NPAD - N), (0, 0)))
    u = _run_prep(x_pad, pos_pad, w1a, w1b, b1.reshape(1, D))

    g = _run_sc_gather(u, nbrT.reshape(E))

    out_pad = _run_mlp(kiter, g.reshape(NKB, KB, QPAD, D), v, cntf,
                       W2.astype(jnp.bfloat16), b2.reshape(1, D),
                       W3.astype(jnp.bfloat16), b3.reshape(1, DOUT))

    out = out_pad[:Q]
    batch_q = jnp.take(batch, sel, axis=0)
    return out, pos_q, batch_q
